# Initial kernel scaffold; baseline (speedup 1.0000x reference)
import jax
import jax.numpy as jnp
from jax.experimental import pallas as pl


def kernel(Z_t, W1, b1, W2, b2):
    raise NotImplementedError("write your pallas kernel here")


# speedup vs baseline: 3.8602x; 3.8602x over previous
import functools

import jax
import jax.numpy as jnp
from jax.experimental import pallas as pl
from jax.experimental.pallas import tpu as pltpu

N = 4096
E = 256
H = 128
K = 16
BR = 256
GRID = N // BR


def _graph_kernel(Z_ref, W1T_ref, b1_ref, W2T_ref, b2_ref,
                  zhat_ref, anorm_ref, asp_ref, loss_ref):
    i = pl.program_id(0)
    Zb = Z_ref[pl.ds(i * BR, BR), :]
    h = jnp.maximum(
        jnp.dot(Zb, W1T_ref[...], preferred_element_type=jnp.float32)
        + b1_ref[...], 0.0)
    A = jnp.abs(
        jnp.dot(h, W2T_ref[...], preferred_element_type=jnp.float32)
        + b2_ref[...])
    mn = jnp.min(A, axis=1, keepdims=True)
    mx = jnp.max(A, axis=1, keepdims=True)
    An = (A - mn) / (mx - mn + 1e-8)
    anorm_ref[...] = An

    col = jax.lax.broadcasted_iota(jnp.int32, (BR, N), 1)
    work = An
    for _ in range(K):
        m = jnp.max(work, axis=1, keepdims=True)
        idx = jnp.min(jnp.where(work == m, col, N), axis=1, keepdims=True)
        work = jnp.where(col == idx, -1.0, work)
    Asp = jnp.where(work < 0.0, An, 0.0)
    asp_ref[...] = Asp

    Zh = jnp.dot(Asp, Z_ref[...], preferred_element_type=jnp.float32)
    zhat_ref[...] = Zh
    loss_ref[...] = jnp.sum((Zh - Zb) ** 2).reshape(1, 1, 1)


@jax.jit
def kernel(Z_t, W1, b1, W2, b2):
    W1T = W1.T
    W2T = W2.T
    b1r = b1.reshape(1, H)
    b2r = b2.reshape(1, N)

    out_shapes = (
        jax.ShapeDtypeStruct((N, E), jnp.float32),
        jax.ShapeDtypeStruct((N, N), jnp.float32),
        jax.ShapeDtypeStruct((N, N), jnp.float32),
        jax.ShapeDtypeStruct((GRID, 1, 1), jnp.float32),
    )
    full = lambda i: (0, 0)
    grid_spec = pl.GridSpec(
        grid=(GRID,),
        in_specs=[
            pl.BlockSpec((N, E), full),
            pl.BlockSpec((E, H), full),
            pl.BlockSpec((1, H), full),
            pl.BlockSpec((H, N), full),
            pl.BlockSpec((1, N), full),
        ],
        out_specs=(
            pl.BlockSpec((BR, E), lambda i: (i, 0)),
            pl.BlockSpec((BR, N), lambda i: (i, 0)),
            pl.BlockSpec((BR, N), lambda i: (i, 0)),
            pl.BlockSpec((1, 1, 1), lambda i: (i, 0, 0)),
        ),
    )
    Z_hat, A_norm, A_sparse, loss_parts = pl.pallas_call(
        _graph_kernel,
        grid_spec=grid_spec,
        out_shape=out_shapes,
        compiler_params=pltpu.CompilerParams(
            dimension_semantics=("parallel",),
        ),
    )(Z_t, W1T, b1r, W2T, b2r)

    L = jnp.sum(loss_parts) / (N * E)
    L = L.reshape(())
    zero = jnp.zeros((), jnp.float32)
    return (Z_hat, A_norm, A_sparse, A_sparse, L, zero, zero, zero)


# speedup vs baseline: 5.8539x; 1.5165x over previous
import functools

import jax
import jax.numpy as jnp
from jax.experimental import pallas as pl
from jax.experimental.pallas import tpu as pltpu

N = 4096
E = 256
H = 128
K = 16
BR = 256
GRID = N // BR


def _graph_kernel(Z_ref, W1T_ref, b1_ref, W2T_ref, b2_ref,
                  zhat_ref, anorm_ref, asp_ref, loss_ref):
    i = pl.program_id(0)
    Zb = Z_ref[pl.ds(i * BR, BR), :]
    h = jnp.maximum(
        jnp.dot(Zb, W1T_ref[...], preferred_element_type=jnp.float32)
        + b1_ref[...], 0.0)
    A = jnp.abs(
        jnp.dot(h, W2T_ref[...], preferred_element_type=jnp.float32)
        + b2_ref[...])
    mn = jnp.min(A, axis=1, keepdims=True)
    mx = jnp.max(A, axis=1, keepdims=True)
    An = (A - mn) / (mx - mn + 1e-8)
    anorm_ref[...] = An

    work = An
    for _ in range(K):
        m = jnp.max(work, axis=1, keepdims=True)
        work = jnp.where(work == m, -1.0, work)
    sel = work < 0.0
    nsel = jnp.sum(sel.astype(jnp.float32), axis=1)
    exact = jnp.all(nsel == float(K))

    Asp = jnp.where(sel, An, 0.0)
    asp_ref[...] = Asp
    Zh = jnp.dot(Asp, Z_ref[...], preferred_element_type=jnp.float32)
    zhat_ref[...] = Zh
    loss_ref[...] = jnp.sum((Zh - Zb) ** 2).reshape(1, 1, 1)

    @pl.when(jnp.logical_not(exact))
    def _fallback():
        col = jax.lax.broadcasted_iota(jnp.int32, (BR, N), 1)
        w2 = An
        for _ in range(K):
            m2 = jnp.max(w2, axis=1, keepdims=True)
            idx = jnp.min(jnp.where(w2 == m2, col, N), axis=1, keepdims=True)
            w2 = jnp.where(col == idx, -1.0, w2)
        Asp2 = jnp.where(w2 < 0.0, An, 0.0)
        asp_ref[...] = Asp2
        Zh2 = jnp.dot(Asp2, Z_ref[...], preferred_element_type=jnp.float32)
        zhat_ref[...] = Zh2
        loss_ref[...] = jnp.sum((Zh2 - Zb) ** 2).reshape(1, 1, 1)


@jax.jit
def kernel(Z_t, W1, b1, W2, b2):
    W1T = W1.T
    W2T = W2.T
    b1r = b1.reshape(1, H)
    b2r = b2.reshape(1, N)

    out_shapes = (
        jax.ShapeDtypeStruct((N, E), jnp.float32),
        jax.ShapeDtypeStruct((N, N), jnp.float32),
        jax.ShapeDtypeStruct((N, N), jnp.float32),
        jax.ShapeDtypeStruct((GRID, 1, 1), jnp.float32),
    )
    full = lambda i: (0, 0)
    grid_spec = pl.GridSpec(
        grid=(GRID,),
        in_specs=[
            pl.BlockSpec((N, E), full),
            pl.BlockSpec((E, H), full),
            pl.BlockSpec((1, H), full),
            pl.BlockSpec((H, N), full),
            pl.BlockSpec((1, N), full),
        ],
        out_specs=(
            pl.BlockSpec((BR, E), lambda i: (i, 0)),
            pl.BlockSpec((BR, N), lambda i: (i, 0)),
            pl.BlockSpec((BR, N), lambda i: (i, 0)),
            pl.BlockSpec((1, 1, 1), lambda i: (i, 0, 0)),
        ),
    )
    Z_hat, A_norm, A_sparse, loss_parts = pl.pallas_call(
        _graph_kernel,
        grid_spec=grid_spec,
        out_shape=out_shapes,
        compiler_params=pltpu.CompilerParams(
            dimension_semantics=("parallel",),
        ),
    )(Z_t, W1T, b1r, W2T, b2r)

    L = jnp.sum(loss_parts) / (N * E)
    L = L.reshape(())
    zero = jnp.zeros((), jnp.float32)
    return (Z_hat, A_norm, A_sparse, A_sparse, L, zero, zero, zero)


# speedup vs baseline: 7.7735x; 1.3279x over previous
import functools

import jax
import jax.numpy as jnp
from jax.experimental import pallas as pl
from jax.experimental.pallas import tpu as pltpu

N = 4096
E = 256
H = 128
K = 16
BR = 256
GRID = N // BR


def _graph_kernel(Z_ref, W1T_ref, b1_ref, W2T_ref, b2_ref,
                  zhat_ref, anorm_ref, asp_ref, loss_ref):
    i = pl.program_id(0)
    Zb = Z_ref[pl.ds(i * BR, BR), :]
    h = jnp.maximum(
        jnp.dot(Zb, W1T_ref[...], preferred_element_type=jnp.float32)
        + b1_ref[...], 0.0)
    A = jnp.abs(
        jnp.dot(h, W2T_ref[...], preferred_element_type=jnp.float32)
        + b2_ref[...])
    mn = jnp.min(A, axis=1, keepdims=True)
    mx = jnp.max(A, axis=1, keepdims=True)
    An = (A - mn) / (mx - mn + 1e-8)
    anorm_ref[...] = An

    neg = jnp.full((BR, 128), -1.0, dtype=jnp.float32)
    M1 = M2 = M3 = M4 = neg
    for c in range(N // 128):
        v = An[:, c * 128:(c + 1) * 128]
        b1_ = jnp.minimum(M1, v)
        M1 = jnp.maximum(M1, v)
        b2_ = jnp.minimum(M2, b1_)
        M2 = jnp.maximum(M2, b1_)
        b3_ = jnp.minimum(M3, b2_)
        M3 = jnp.maximum(M3, b2_)
        M4 = jnp.maximum(M4, b3_)
    Mc = jnp.concatenate([M1, M2, M3, M4], axis=1)
    m = mx * 0.0 + 2.0
    for _ in range(K):
        m = jnp.max(jnp.where(Mc < m, Mc, -1.0), axis=1, keepdims=True)
    sel = An >= m
    nsel = jnp.sum(sel.astype(jnp.float32), axis=1)
    exact = jnp.all(nsel == float(K))

    Asp = jnp.where(sel, An, 0.0)
    asp_ref[...] = Asp
    Zh = jnp.dot(Asp, Z_ref[...], preferred_element_type=jnp.float32)
    zhat_ref[...] = Zh
    loss_ref[...] = jnp.sum((Zh - Zb) ** 2).reshape(1, 1, 1)

    @pl.when(jnp.logical_not(exact))
    def _fallback():
        col = jax.lax.broadcasted_iota(jnp.int32, (BR, N), 1)
        w2 = An
        for _ in range(K):
            m2 = jnp.max(w2, axis=1, keepdims=True)
            idx = jnp.min(jnp.where(w2 == m2, col, N), axis=1, keepdims=True)
            w2 = jnp.where(col == idx, -1.0, w2)
        Asp2 = jnp.where(w2 < 0.0, An, 0.0)
        asp_ref[...] = Asp2
        Zh2 = jnp.dot(Asp2, Z_ref[...], preferred_element_type=jnp.float32)
        zhat_ref[...] = Zh2
        loss_ref[...] = jnp.sum((Zh2 - Zb) ** 2).reshape(1, 1, 1)


@jax.jit
def kernel(Z_t, W1, b1, W2, b2):
    W1T = W1.T
    W2T = W2.T
    b1r = b1.reshape(1, H)
    b2r = b2.reshape(1, N)

    out_shapes = (
        jax.ShapeDtypeStruct((N, E), jnp.float32),
        jax.ShapeDtypeStruct((N, N), jnp.float32),
        jax.ShapeDtypeStruct((N, N), jnp.float32),
        jax.ShapeDtypeStruct((GRID, 1, 1), jnp.float32),
    )
    full = lambda i: (0, 0)
    grid_spec = pl.GridSpec(
        grid=(GRID,),
        in_specs=[
            pl.BlockSpec((N, E), full),
            pl.BlockSpec((E, H), full),
            pl.BlockSpec((1, H), full),
            pl.BlockSpec((H, N), full),
            pl.BlockSpec((1, N), full),
        ],
        out_specs=(
            pl.BlockSpec((BR, E), lambda i: (i, 0)),
            pl.BlockSpec((BR, N), lambda i: (i, 0)),
            pl.BlockSpec((BR, N), lambda i: (i, 0)),
            pl.BlockSpec((1, 1, 1), lambda i: (i, 0, 0)),
        ),
    )
    Z_hat, A_norm, A_sparse, loss_parts = pl.pallas_call(
        _graph_kernel,
        grid_spec=grid_spec,
        out_shape=out_shapes,
        compiler_params=pltpu.CompilerParams(
            dimension_semantics=("parallel",),
        ),
    )(Z_t, W1T, b1r, W2T, b2r)

    L = jnp.sum(loss_parts) / (N * E)
    L = L.reshape(())
    zero = jnp.zeros((), jnp.float32)
    return (Z_hat, A_norm, A_sparse, A_sparse, L, zero, zero, zero)
